# baseline (speedup 1.0000x reference)
import functools

import jax
import jax.numpy as jnp
from jax import lax
from jax.experimental import pallas as pl
from jax.experimental.pallas import tpu as pltpu
from jax.experimental.pallas import tpu_sc as plsc

_B = 16384
_NF = 26
_V = 100000
_D = 32
_H = 256
_O = 1

_NC = 2
_NS = 16
_NW = _NC * _NS

_ROWS = _B * _NF
_ROWS_W = _ROWS // _NW
_CHUNK = 128
_NCHUNK = _ROWS_W // _CHUNK
_GROUP = 8
_GROUP_ROWS = _CHUNK * _GROUP
_NGROUP = _NCHUNK // _GROUP


def _make_gather_body(nchunk, ngroup, rows_w):
    def _gather_body(table_hbm, idx_hbm, out_hbm, idx_v, rows_v, sem):
        wid = lax.axis_index("s") * _NC + lax.axis_index("c")
        base = wid * rows_w
        pltpu.sync_copy(idx_hbm.at[wid], idx_v)

        def group(g, carry):
            cbase = g * _GROUP
            copies = []
            for j in range(_GROUP):
                cp = pltpu.async_copy(
                    table_hbm.at[idx_v.at[cbase + j]],
                    rows_v.at[pl.ds(j * _CHUNK, _CHUNK)],
                    sem,
                )
                copies.append(cp)
            for cp in copies:
                cp.wait()
            pltpu.sync_copy(
                rows_v, out_hbm.at[pl.ds(base + g * _GROUP_ROWS, _GROUP_ROWS)]
            )
            return carry

        lax.fori_loop(0, ngroup, group, 0)

    return _gather_body


def _sc_gather(table_flat, idx3, nrows):
    rows_w = nrows // _NW
    nchunk = rows_w // _CHUNK
    ngroup = nchunk // _GROUP
    mesh = plsc.VectorSubcoreMesh(core_axis_name="c", subcore_axis_name="s")
    f = pl.kernel(
        _make_gather_body(nchunk, ngroup, rows_w),
        mesh=mesh,
        out_type=jax.ShapeDtypeStruct((nrows, _D), jnp.float32),
        scratch_types=[
            pltpu.VMEM((nchunk, _CHUNK), jnp.int32),
            pltpu.VMEM((_GROUP_ROWS, _D), jnp.float32),
            pltpu.SemaphoreType.DMA,
        ],
        compiler_params=pltpu.CompilerParams(use_tc_tiling_on_sc=False),
    )
    return f(table_flat, idx3)


_BB = 512


def _mlp_body(*refs):
    e_refs = refs[:_NF // 2]
    w1t_ref, b1_ref, w2t_ref, b2_ref, out_ref = refs[_NF // 2:]
    pieces = []
    for c in range(_NF // 2):
        for j in range(2):
            pjt = jnp.swapaxes(e_refs[c][j], 0, 1)
            pieces.append(jnp.concatenate(
                [pjt[32 * a:32 * (a + 1), :] for a in range(4)], axis=1))
    et = jnp.concatenate(pieces, axis=0)
    h = jnp.dot(w1t_ref[...], et, preferred_element_type=jnp.float32)
    h = jnp.maximum(h + b1_ref[...], 0.0)
    o = jnp.dot(w2t_ref[...], h, preferred_element_type=jnp.float32)
    out_ref[...] = jax.nn.sigmoid(o + b2_ref[...])


def _tc_mlp(embs, W1, b1, W2, b2):
    grid = (_B // _BB,)
    outp = pl.pallas_call(
        _mlp_body,
        grid=grid,
        in_specs=[pl.BlockSpec((2, _BB // 4, 128), lambda i: (0, i, 0))
                  for _ in range(_NF // 2)] + [
            pl.BlockSpec((_H, _NF * _D), lambda i: (0, 0)),
            pl.BlockSpec((_H, 1), lambda i: (0, 0)),
            pl.BlockSpec((_O, _H), lambda i: (0, 0)),
            pl.BlockSpec((_O, 1), lambda i: (0, 0)),
        ],
        out_specs=pl.BlockSpec((1, _BB), lambda i: (0, i)),
        out_shape=jax.ShapeDtypeStruct((1, _B), jnp.float32),
    )(*embs, W1.T, b1.reshape(_H, 1), W2.T, b2.reshape(_O, 1))
    return outp.reshape(_B // _BB, 4, _BB // 4).transpose(0, 2, 1).reshape(_B, _O)


_TB = _V // 4


def _tr_body(t_ref, o_ref):
    s = t_ref[0]
    parts = [s[:, a * _TB:(a + 1) * _TB] for a in range(4)]
    stacked = jnp.concatenate(parts, axis=0)
    o_ref[...] = jnp.swapaxes(stacked, 0, 1)


def _tc_relayout(tabT_c, nf):
    grid = (nf,)
    return pl.pallas_call(
        _tr_body,
        grid=grid,
        in_specs=[pl.BlockSpec((1, _D, _V), lambda f: (f, 0, 0))],
        out_specs=pl.BlockSpec((_TB, 4 * _D), lambda f: (f, 0)),
        out_shape=jax.ShapeDtypeStruct((nf * _V // 4, 4 * _D), jnp.float32),
        compiler_params=pltpu.CompilerParams(vmem_limit_bytes=100 * 1024 * 1024),
    )(tabT_c)


def kernel(x, tables, W1, b1, W2, b2):
    offs = ((jnp.arange(_NF, dtype=jnp.int32) % 2) * _V)[:, None]
    xi = x.astype(jnp.int32)
    perm = 4 * (xi % _TB) + xi // _TB
    idxT = perm.T + offs
    tabT = jnp.transpose(tables, (0, 2, 1))
    embs = []
    for c in range(_NF // 2):
        tf_c = _tc_relayout(lax.slice_in_dim(tabT, 2 * c, 2 * c + 2, axis=0), 2)
        idx_c = idxT[2 * c:2 * c + 2].reshape(_NW, (2 * _B) // (_NW * _CHUNK), _CHUNK)
        emb_c = _sc_gather(tf_c.reshape(2 * _V, _D), idx_c, 2 * _B)
        embs.append(emb_c.reshape(2, _B * _D // 128, 128))
    return _tc_mlp(embs, W1, b1, W2, b2)


# speedup vs baseline: 1.1051x; 1.1051x over previous
import functools

import jax
import jax.numpy as jnp
from jax import lax
from jax.experimental import pallas as pl
from jax.experimental.pallas import tpu as pltpu
from jax.experimental.pallas import tpu_sc as plsc

_B = 16384
_NF = 26
_V = 100000
_D = 32
_H = 256
_O = 1

_NC = 2
_NS = 16
_NW = _NC * _NS

_ROWS = _B * _NF
_ROWS_W = _ROWS // _NW
_CHUNK = 128
_NCHUNK = _ROWS_W // _CHUNK
_GROUP = 8
_GROUP_ROWS = _CHUNK * _GROUP
_NGROUP = _NCHUNK // _GROUP

_FCH = 13
_NCH = _NF // _FCH
_GGRP = 13


def _make_gather_body(nchunk, ngroup, rows_w, group=_GROUP):
    def _gather_body(table_hbm, idx_hbm, out_hbm, idx_v, rows_v, sem):
        wid = lax.axis_index("s") * _NC + lax.axis_index("c")
        base = wid * rows_w
        pltpu.sync_copy(idx_hbm.at[wid], idx_v)

        grows = group * _CHUNK

        def body(g, carry):
            cbase = g * group
            copies = []
            for j in range(group):
                cp = pltpu.async_copy(
                    table_hbm.at[idx_v.at[cbase + j]],
                    rows_v.at[pl.ds(j * _CHUNK, _CHUNK)],
                    sem,
                )
                copies.append(cp)
            for cp in copies:
                cp.wait()
            pltpu.sync_copy(
                rows_v, out_hbm.at[pl.ds(base + g * grows, grows)]
            )
            return carry

        lax.fori_loop(0, ngroup, body, 0)

    return _gather_body


def _sc_gather(table_flat, idx3, nrows, group=_GROUP):
    rows_w = nrows // _NW
    nchunk = rows_w // _CHUNK
    ngroup = nchunk // group
    mesh = plsc.VectorSubcoreMesh(core_axis_name="c", subcore_axis_name="s")
    f = pl.kernel(
        _make_gather_body(nchunk, ngroup, rows_w, group),
        mesh=mesh,
        out_type=jax.ShapeDtypeStruct((nrows, _D), jnp.float32),
        scratch_types=[
            pltpu.VMEM((nchunk, _CHUNK), jnp.int32),
            pltpu.VMEM((group * _CHUNK, _D), jnp.float32),
            pltpu.SemaphoreType.DMA,
        ],
        compiler_params=pltpu.CompilerParams(use_tc_tiling_on_sc=False),
    )
    return f(table_flat, idx3)


_BB = 512


def _mlp_body(*refs):
    e_refs = refs[:_NCH]
    w1t_ref, b1_ref, w2t_ref, b2_ref, out_ref = refs[_NCH:]
    pieces = []
    for c in range(_NCH):
        for j in range(_FCH):
            pjt = jnp.swapaxes(e_refs[c][j], 0, 1)
            pieces.append(jnp.concatenate(
                [pjt[32 * a:32 * (a + 1), :] for a in range(4)], axis=1))
    et = jnp.concatenate(pieces, axis=0)
    h = jnp.dot(w1t_ref[...], et, preferred_element_type=jnp.float32)
    h = jnp.maximum(h + b1_ref[...], 0.0)
    o = jnp.dot(w2t_ref[...], h, preferred_element_type=jnp.float32)
    out_ref[...] = jax.nn.sigmoid(o + b2_ref[...])


def _tc_mlp(embs, W1, b1, W2, b2):
    grid = (_B // _BB,)
    outp = pl.pallas_call(
        _mlp_body,
        grid=grid,
        in_specs=[pl.BlockSpec((_FCH, _BB // 4, 128), lambda i: (0, i, 0))
                  for _ in range(_NCH)] + [
            pl.BlockSpec((_H, _NF * _D), lambda i: (0, 0)),
            pl.BlockSpec((_H, 1), lambda i: (0, 0)),
            pl.BlockSpec((_O, _H), lambda i: (0, 0)),
            pl.BlockSpec((_O, 1), lambda i: (0, 0)),
        ],
        out_specs=pl.BlockSpec((1, _BB), lambda i: (0, i)),
        out_shape=jax.ShapeDtypeStruct((1, _B), jnp.float32),
    )(*embs, W1.T, b1.reshape(_H, 1), W2.T, b2.reshape(_O, 1))
    return outp.reshape(_B // _BB, 4, _BB // 4).transpose(0, 2, 1).reshape(_B, _O)


_TB = _V // 4


def _tr_body(t_ref, o_ref):
    s = t_ref[0]
    parts = [s[:, a * _TB:(a + 1) * _TB] for a in range(4)]
    stacked = jnp.concatenate(parts, axis=0)
    o_ref[...] = jnp.swapaxes(stacked, 0, 1)


def _tc_relayout(tabT_c, nf):
    grid = (nf,)
    return pl.pallas_call(
        _tr_body,
        grid=grid,
        in_specs=[pl.BlockSpec((1, _D, _V), lambda f: (f, 0, 0))],
        out_specs=pl.BlockSpec((_TB, 4 * _D), lambda f: (f, 0)),
        out_shape=jax.ShapeDtypeStruct((nf * _V // 4, 4 * _D), jnp.float32),
        compiler_params=pltpu.CompilerParams(vmem_limit_bytes=100 * 1024 * 1024),
    )(tabT_c)


def kernel(x, tables, W1, b1, W2, b2):
    offs = ((jnp.arange(_NF, dtype=jnp.int32) % _FCH) * _V)[:, None]
    xi = x.astype(jnp.int32)
    perm = 4 * (xi % _TB) + xi // _TB
    idxT = perm.T + offs
    tabT = jnp.transpose(tables, (0, 2, 1))
    embs = []
    for c in range(_NCH):
        lo = c * _FCH
        tf_c = _tc_relayout(lax.slice_in_dim(tabT, lo, lo + _FCH, axis=0), _FCH)
        nrows = _FCH * _B
        idx_c = idxT[lo:lo + _FCH].reshape(_NW, nrows // (_NW * _CHUNK), _CHUNK)
        emb_c = _sc_gather(tf_c.reshape(_FCH * _V, _D), idx_c, nrows, _GGRP)
        embs.append(emb_c.reshape(_FCH, _B * _D // 128, 128))
    return _tc_mlp(embs, W1, b1, W2, b2)


# speedup vs baseline: 1.5366x; 1.3905x over previous
import functools

import jax
import jax.numpy as jnp
from jax import lax
from jax.experimental import pallas as pl
from jax.experimental.pallas import tpu as pltpu
from jax.experimental.pallas import tpu_sc as plsc

_B = 16384
_NF = 26
_V = 100000
_D = 32
_H = 256
_O = 1

_NC = 2
_NS = 16
_NW = _NC * _NS

_ROWS = _B * _NF
_ROWS_W = _ROWS // _NW
_CHUNK = 128
_NCHUNK = _ROWS_W // _CHUNK
_GROUP = 8
_GROUP_ROWS = _CHUNK * _GROUP
_NGROUP = _NCHUNK // _GROUP

_FCH = 2
_NCH = _NF // _FCH
_GGRP = 8


def _make_gather_body(nchunk, ngroup, rows_w, group=_GROUP):
    def _gather_body(table_hbm, idx_hbm, out_hbm, idx_v, rows_v, sem):
        wid = lax.axis_index("s") * _NC + lax.axis_index("c")
        base = wid * rows_w
        pltpu.sync_copy(idx_hbm.at[wid], idx_v)

        grows = group * _CHUNK

        def body(g, carry):
            cbase = g * group
            copies = []
            for j in range(group):
                cp = pltpu.async_copy(
                    table_hbm.at[idx_v.at[cbase + j]],
                    rows_v.at[pl.ds(j * _CHUNK, _CHUNK)],
                    sem,
                )
                copies.append(cp)
            for cp in copies:
                cp.wait()
            pltpu.sync_copy(
                rows_v, out_hbm.at[pl.ds(base + g * grows, grows)]
            )
            return carry

        lax.fori_loop(0, ngroup, body, 0)

    return _gather_body


def _sc_gather(table_flat, idx3, nrows, group=_GROUP):
    rows_w = nrows // _NW
    nchunk = rows_w // _CHUNK
    ngroup = nchunk // group
    mesh = plsc.VectorSubcoreMesh(core_axis_name="c", subcore_axis_name="s")
    f = pl.kernel(
        _make_gather_body(nchunk, ngroup, rows_w, group),
        mesh=mesh,
        out_type=jax.ShapeDtypeStruct((nrows, _D), jnp.float32),
        scratch_types=[
            pltpu.VMEM((nchunk, _CHUNK), jnp.int32),
            pltpu.VMEM((group * _CHUNK, _D), jnp.float32),
            pltpu.SemaphoreType.DMA,
        ],
        compiler_params=pltpu.CompilerParams(use_tc_tiling_on_sc=False),
    )
    return f(table_flat, idx3)


_BB = 512


def _mlp_body(*refs):
    e_refs = refs[:_NCH]
    w1t_ref, b1_ref, w2t_ref, b2_ref, out_ref = refs[_NCH:]
    pieces = []
    for c in range(_NCH):
        for j in range(_FCH):
            pjt = jnp.swapaxes(e_refs[c][j], 0, 1)
            pieces.append(jnp.concatenate(
                [pjt[32 * a:32 * (a + 1), :] for a in range(4)], axis=1))
    et = jnp.concatenate(pieces, axis=0)
    h = jnp.dot(w1t_ref[...], et, preferred_element_type=jnp.float32)
    h = jnp.maximum(h + b1_ref[...], 0.0)
    o = jnp.dot(w2t_ref[...], h, preferred_element_type=jnp.float32)
    out_ref[...] = jax.nn.sigmoid(o + b2_ref[...])


def _tc_mlp(embs, W1, b1, W2, b2):
    grid = (_B // _BB,)
    outp = pl.pallas_call(
        _mlp_body,
        grid=grid,
        in_specs=[pl.BlockSpec((_FCH, _BB // 4, 128), lambda i: (0, i, 0))
                  for _ in range(_NCH)] + [
            pl.BlockSpec((_H, _NF * _D), lambda i: (0, 0)),
            pl.BlockSpec((_H, 1), lambda i: (0, 0)),
            pl.BlockSpec((_O, _H), lambda i: (0, 0)),
            pl.BlockSpec((_O, 1), lambda i: (0, 0)),
        ],
        out_specs=pl.BlockSpec((1, _BB), lambda i: (0, i)),
        out_shape=jax.ShapeDtypeStruct((1, _B), jnp.float32),
    )(*embs, W1.T, b1.reshape(_H, 1), W2.T, b2.reshape(_O, 1))
    return outp.reshape(_B // _BB, 4, _BB // 4).transpose(0, 2, 1).reshape(_B, _O)


_TB = _V // 4


def _tr_body(t_ref, o_ref):
    s = t_ref[0]
    parts = [s[:, a * _TB:(a + 1) * _TB] for a in range(4)]
    stacked = jnp.concatenate(parts, axis=0)
    o_ref[...] = jnp.swapaxes(stacked, 0, 1)


def _tc_relayout(tabT_c, nf, lo=0):
    grid = (nf,)
    return pl.pallas_call(
        _tr_body,
        grid=grid,
        in_specs=[pl.BlockSpec((1, _D, _V), lambda f, lo=lo: (lo + f, 0, 0))],
        out_specs=pl.BlockSpec((_TB, 4 * _D), lambda f: (f, 0)),
        out_shape=jax.ShapeDtypeStruct((nf * _V // 4, 4 * _D), jnp.float32),
        compiler_params=pltpu.CompilerParams(vmem_limit_bytes=100 * 1024 * 1024),
    )(tabT_c)


def kernel(x, tables, W1, b1, W2, b2):
    offs = ((jnp.arange(_NF, dtype=jnp.int32) % _FCH) * _V)[:, None]
    xi = x.astype(jnp.int32)
    perm = 4 * (xi % _TB) + xi // _TB
    idxT = perm.T + offs
    tabT = jnp.transpose(tables, (0, 2, 1))
    embs = []
    for c in range(_NCH):
        lo = c * _FCH
        tf_c = _tc_relayout(tabT, _FCH, lo)
        nrows = _FCH * _B
        idx_c = idxT[lo:lo + _FCH].reshape(_NW, nrows // (_NW * _CHUNK), _CHUNK)
        emb_c = _sc_gather(tf_c.reshape(_FCH * _V, _D), idx_c, nrows, _GGRP)
        embs.append(emb_c.reshape(_FCH, _B * _D // 128, 128))
    return _tc_mlp(embs, W1, b1, W2, b2)


# speedup vs baseline: 1.8215x; 1.1854x over previous
import functools

import jax
import jax.numpy as jnp
from jax import lax
from jax.experimental import pallas as pl
from jax.experimental.pallas import tpu as pltpu
from jax.experimental.pallas import tpu_sc as plsc

_B = 16384
_NF = 26
_V = 100000
_D = 32
_H = 256
_O = 1

_NC = 2
_NS = 16
_NW = _NC * _NS

_ROWS = _B * _NF
_ROWS_W = _ROWS // _NW
_CHUNK = 128
_NCHUNK = _ROWS_W // _CHUNK
_GROUP = 8
_GROUP_ROWS = _CHUNK * _GROUP
_NGROUP = _NCHUNK // _GROUP

_FCH = 13
_NCH = _NF // _FCH
_GGRP = 13


def _make_gather_body(nchunk, ngroup, rows_w, group=_GROUP):
    def _gather_body(table_hbm, idx_hbm, out_hbm, idx_v, rows_v, sem):
        wid = lax.axis_index("s") * _NC + lax.axis_index("c")
        base = wid * rows_w
        pltpu.sync_copy(idx_hbm.at[wid], idx_v)

        grows = group * _CHUNK

        def body(g, carry):
            cbase = g * group
            copies = []
            for j in range(group):
                cp = pltpu.async_copy(
                    table_hbm.at[idx_v.at[cbase + j]],
                    rows_v.at[pl.ds(j * _CHUNK, _CHUNK)],
                    sem,
                )
                copies.append(cp)
            for cp in copies:
                cp.wait()
            pltpu.sync_copy(
                rows_v, out_hbm.at[pl.ds(base + g * grows, grows)]
            )
            return carry

        lax.fori_loop(0, ngroup, body, 0)

    return _gather_body


def _sc_gather(table_flat, idx3, nrows, group=_GROUP):
    rows_w = nrows // _NW
    nchunk = rows_w // _CHUNK
    ngroup = nchunk // group
    mesh = plsc.VectorSubcoreMesh(core_axis_name="c", subcore_axis_name="s")
    f = pl.kernel(
        _make_gather_body(nchunk, ngroup, rows_w, group),
        mesh=mesh,
        out_type=jax.ShapeDtypeStruct((nrows, _D), jnp.float32),
        scratch_types=[
            pltpu.VMEM((nchunk, _CHUNK), jnp.int32),
            pltpu.VMEM((group * _CHUNK, _D), jnp.float32),
            pltpu.SemaphoreType.DMA,
        ],
        compiler_params=pltpu.CompilerParams(use_tc_tiling_on_sc=False),
    )
    return f(table_flat, idx3)


_BB = 512


def _mlp_body(*refs):
    e_refs = refs[:_NCH]
    w1t_ref, b1_ref, w2t_ref, b2_ref, out_ref = refs[_NCH:]
    pieces = []
    for c in range(_NCH):
        for j in range(_FCH):
            pjt = jnp.swapaxes(e_refs[c][j], 0, 1)
            pieces.append(jnp.concatenate(
                [pjt[32 * a:32 * (a + 1), :] for a in range(4)], axis=1))
    et = jnp.concatenate(pieces, axis=0)
    h = jnp.dot(w1t_ref[...], et, preferred_element_type=jnp.float32)
    h = jnp.maximum(h + b1_ref[...], 0.0)
    o = jnp.dot(w2t_ref[...], h, preferred_element_type=jnp.float32)
    out_ref[...] = jax.nn.sigmoid(o + b2_ref[...])


def _tc_mlp(embs, W1, b1, W2, b2):
    grid = (_B // _BB,)
    outp = pl.pallas_call(
        _mlp_body,
        grid=grid,
        in_specs=[pl.BlockSpec((_FCH, _BB // 4, 128), lambda i: (0, i, 0))
                  for _ in range(_NCH)] + [
            pl.BlockSpec((_H, _NF * _D), lambda i: (0, 0)),
            pl.BlockSpec((_H, 1), lambda i: (0, 0)),
            pl.BlockSpec((_O, _H), lambda i: (0, 0)),
            pl.BlockSpec((_O, 1), lambda i: (0, 0)),
        ],
        out_specs=pl.BlockSpec((1, _BB), lambda i: (0, i)),
        out_shape=jax.ShapeDtypeStruct((1, _B), jnp.float32),
    )(*embs, W1.T, b1.reshape(_H, 1), W2.T, b2.reshape(_O, 1))
    return outp.reshape(_B // _BB, 4, _BB // 4).transpose(0, 2, 1).reshape(_B, _O)


_TB = _V // 4


def _tr_body(t_ref, o_ref):
    s = t_ref[0]
    parts = [s[:, a * _TB:(a + 1) * _TB] for a in range(4)]
    stacked = jnp.concatenate(parts, axis=0)
    o_ref[...] = jnp.swapaxes(stacked, 0, 1)


def _tc_relayout(tabT_c, nf, lo=0):
    grid = (nf,)
    return pl.pallas_call(
        _tr_body,
        grid=grid,
        in_specs=[pl.BlockSpec((1, _D, _V), lambda f, lo=lo: (lo + f, 0, 0))],
        out_specs=pl.BlockSpec((_TB, 4 * _D), lambda f: (f, 0)),
        out_shape=jax.ShapeDtypeStruct((nf * _V // 4, 4 * _D), jnp.float32),
        compiler_params=pltpu.CompilerParams(vmem_limit_bytes=100 * 1024 * 1024),
    )(tabT_c)


def kernel(x, tables, W1, b1, W2, b2):
    offs = ((jnp.arange(_NF, dtype=jnp.int32) % _FCH) * _V)[:, None]
    xi = x.astype(jnp.int32)
    perm = 4 * (xi % _TB) + xi // _TB
    idxT = perm.T + offs
    tabT = jnp.transpose(tables, (0, 2, 1))
    embs = []
    for c in range(_NCH):
        lo = c * _FCH
        tf_c = _tc_relayout(tabT, _FCH, lo)
        nrows = _FCH * _B
        idx_c = idxT[lo:lo + _FCH].reshape(_NW, nrows // (_NW * _CHUNK), _CHUNK)
        emb_c = _sc_gather(tf_c.reshape(_FCH * _V, _D), idx_c, nrows, _GGRP)
        embs.append(emb_c.reshape(_FCH, _B * _D // 128, 128))
    return _tc_mlp(embs, W1, b1, W2, b2)


# speedup vs baseline: 1.8284x; 1.0038x over previous
import functools

import jax
import jax.numpy as jnp
import numpy as np
from jax import lax
from jax.experimental import pallas as pl
from jax.experimental.pallas import tpu as pltpu
from jax.experimental.pallas import tpu_sc as plsc

_B = 16384
_NF = 26
_V = 100000
_D = 32
_H = 256
_O = 1

_NC = 2
_NS = 16
_NW = _NC * _NS

_ROWS = _B * _NF
_ROWS_W = _ROWS // _NW
_CHUNK = 128
_NCHUNK = _ROWS_W // _CHUNK
_GROUP = 8
_GROUP_ROWS = _CHUNK * _GROUP
_NGROUP = _NCHUNK // _GROUP

_CHUNKS = (20, 6)
_GGRP = 8


def _make_gather_body(nchunk, ngroup, rows_w, group=_GROUP):
    def _gather_body(table_hbm, idx_hbm, out_hbm, idx_v, rows_v, sem):
        wid = lax.axis_index("s") * _NC + lax.axis_index("c")
        base = wid * rows_w
        pltpu.sync_copy(idx_hbm.at[wid], idx_v)

        grows = group * _CHUNK

        def body(g, carry):
            cbase = g * group
            copies = []
            for j in range(group):
                cp = pltpu.async_copy(
                    table_hbm.at[idx_v.at[cbase + j]],
                    rows_v.at[pl.ds(j * _CHUNK, _CHUNK)],
                    sem,
                )
                copies.append(cp)
            for cp in copies:
                cp.wait()
            pltpu.sync_copy(
                rows_v, out_hbm.at[pl.ds(base + g * grows, grows)]
            )
            return carry

        lax.fori_loop(0, ngroup, body, 0)

    return _gather_body


def _sc_gather(table_flat, idx3, nrows, group=_GROUP):
    rows_w = nrows // _NW
    nchunk = rows_w // _CHUNK
    ngroup = nchunk // group
    mesh = plsc.VectorSubcoreMesh(core_axis_name="c", subcore_axis_name="s")
    f = pl.kernel(
        _make_gather_body(nchunk, ngroup, rows_w, group),
        mesh=mesh,
        out_type=jax.ShapeDtypeStruct((nrows, _D), jnp.float32),
        scratch_types=[
            pltpu.VMEM((nchunk, _CHUNK), jnp.int32),
            pltpu.VMEM((group * _CHUNK, _D), jnp.float32),
            pltpu.SemaphoreType.DMA,
        ],
        compiler_params=pltpu.CompilerParams(use_tc_tiling_on_sc=False),
    )
    return f(table_flat, idx3)


_BB = 512


def _mlp_body(*refs):
    nch = len(_CHUNKS)
    e_refs = refs[:nch]
    w1t_ref, b1_ref, w2t_ref, b2_ref, out_ref = refs[nch:]
    pieces = []
    for c in range(nch):
        for j in range(_CHUNKS[c]):
            pjt = jnp.swapaxes(e_refs[c][j], 0, 1)
            pieces.append(jnp.concatenate(
                [pjt[32 * a:32 * (a + 1), :] for a in range(4)], axis=1))
    et = jnp.concatenate(pieces, axis=0)
    h = jnp.dot(w1t_ref[...], et, preferred_element_type=jnp.float32)
    h = jnp.maximum(h + b1_ref[...], 0.0)
    o = jnp.dot(w2t_ref[...], h, preferred_element_type=jnp.float32)
    out_ref[...] = jax.nn.sigmoid(o + b2_ref[...])


def _tc_mlp(embs, W1, b1, W2, b2):
    grid = (_B // _BB,)
    outp = pl.pallas_call(
        _mlp_body,
        grid=grid,
        in_specs=[pl.BlockSpec((n, _BB // 4, 128), lambda i: (0, i, 0))
                  for n in _CHUNKS] + [
            pl.BlockSpec((_H, _NF * _D), lambda i: (0, 0)),
            pl.BlockSpec((_H, 1), lambda i: (0, 0)),
            pl.BlockSpec((_O, _H), lambda i: (0, 0)),
            pl.BlockSpec((_O, 1), lambda i: (0, 0)),
        ],
        out_specs=pl.BlockSpec((1, _BB), lambda i: (0, i)),
        out_shape=jax.ShapeDtypeStruct((1, _B), jnp.float32),
    )(*embs, W1.T, b1.reshape(_H, 1), W2.T, b2.reshape(_O, 1))
    return outp.reshape(_B // _BB, 4, _BB // 4).transpose(0, 2, 1).reshape(_B, _O)


_TB = _V // 4


def _tr_body(t_ref, o_ref):
    s = t_ref[0]
    parts = [s[:, a * _TB:(a + 1) * _TB] for a in range(4)]
    stacked = jnp.concatenate(parts, axis=0)
    o_ref[...] = jnp.swapaxes(stacked, 0, 1)


def _tc_relayout(tabT_c, nf, lo=0):
    grid = (nf,)
    return pl.pallas_call(
        _tr_body,
        grid=grid,
        in_specs=[pl.BlockSpec((1, _D, _V), lambda f, lo=lo: (lo + f, 0, 0))],
        out_specs=pl.BlockSpec((_TB, 4 * _D), lambda f: (f, 0)),
        out_shape=jax.ShapeDtypeStruct((nf * _V // 4, 4 * _D), jnp.float32),
        compiler_params=pltpu.CompilerParams(vmem_limit_bytes=100 * 1024 * 1024),
    )(tabT_c)


def kernel(x, tables, W1, b1, W2, b2):
    f_local = np.concatenate([np.arange(n) for n in _CHUNKS])
    offs = jnp.asarray(f_local * _V, dtype=jnp.int32)[:, None]
    xi = x.astype(jnp.int32)
    perm = 4 * (xi % _TB) + xi // _TB
    idxT = perm.T + offs
    tabT = jnp.transpose(tables, (0, 2, 1))
    embs = []
    lo = 0
    for n in _CHUNKS:
        tf_c = _tc_relayout(tabT, n, lo)
        nrows = n * _B
        idx_c = idxT[lo:lo + n].reshape(_NW, nrows // (_NW * _CHUNK), _CHUNK)
        emb_c = _sc_gather(tf_c.reshape(n * _V, _D), idx_c, nrows, _GGRP)
        embs.append(emb_c.reshape(n, _B * _D // 128, 128))
        lo += n
    return _tc_mlp(embs, W1, b1, W2, b2)


# speedup vs baseline: 1.8368x; 1.0046x over previous
import functools

import jax
import jax.numpy as jnp
import numpy as np
from jax import lax
from jax.experimental import pallas as pl
from jax.experimental.pallas import tpu as pltpu
from jax.experimental.pallas import tpu_sc as plsc

_B = 16384
_NF = 26
_V = 100000
_D = 32
_H = 256
_O = 1

_NC = 2
_NS = 16
_NW = _NC * _NS

_ROWS = _B * _NF
_ROWS_W = _ROWS // _NW
_CHUNK = 128
_NCHUNK = _ROWS_W // _CHUNK
_GROUP = 8
_GROUP_ROWS = _CHUNK * _GROUP
_NGROUP = _NCHUNK // _GROUP

_CHUNKS = (20, 6)
_GRPS = (16, 24)


def _make_gather_body(nchunk, ngroup, rows_w, group=_GROUP):
    def _gather_body(table_hbm, idx_hbm, out_hbm, idx_v, rows_v, sem):
        wid = lax.axis_index("s") * _NC + lax.axis_index("c")
        base = wid * rows_w
        pltpu.sync_copy(idx_hbm.at[wid], idx_v)

        grows = group * _CHUNK

        def body(g, carry):
            cbase = g * group
            copies = []
            for j in range(group):
                cp = pltpu.async_copy(
                    table_hbm.at[idx_v.at[cbase + j]],
                    rows_v.at[pl.ds(j * _CHUNK, _CHUNK)],
                    sem,
                )
                copies.append(cp)
            for cp in copies:
                cp.wait()
            pltpu.sync_copy(
                rows_v, out_hbm.at[pl.ds(base + g * grows, grows)]
            )
            return carry

        lax.fori_loop(0, ngroup, body, 0)

    return _gather_body


def _sc_gather(table_flat, idx3, nrows, group=_GROUP):
    rows_w = nrows // _NW
    nchunk = rows_w // _CHUNK
    ngroup = nchunk // group
    mesh = plsc.VectorSubcoreMesh(core_axis_name="c", subcore_axis_name="s")
    f = pl.kernel(
        _make_gather_body(nchunk, ngroup, rows_w, group),
        mesh=mesh,
        out_type=jax.ShapeDtypeStruct((nrows, _D), jnp.float32),
        scratch_types=[
            pltpu.VMEM((nchunk, _CHUNK), jnp.int32),
            pltpu.VMEM((group * _CHUNK, _D), jnp.float32),
            pltpu.SemaphoreType.DMA,
        ],
        compiler_params=pltpu.CompilerParams(use_tc_tiling_on_sc=False),
    )
    return f(table_flat, idx3)


_BB = 512


def _mlp_body(*refs):
    nch = len(_CHUNKS)
    e_refs = refs[:nch]
    w1t_ref, b1_ref, w2t_ref, b2_ref, out_ref = refs[nch:]
    pieces = []
    for c in range(nch):
        for j in range(_CHUNKS[c]):
            pjt = jnp.swapaxes(e_refs[c][j], 0, 1)
            pieces.append(jnp.concatenate(
                [pjt[32 * a:32 * (a + 1), :] for a in range(4)], axis=1))
    et = jnp.concatenate(pieces, axis=0)
    h = jnp.dot(w1t_ref[...], et, preferred_element_type=jnp.float32)
    h = jnp.maximum(h + b1_ref[...], 0.0)
    o = jnp.dot(w2t_ref[...], h, preferred_element_type=jnp.float32)
    out_ref[...] = jax.nn.sigmoid(o + b2_ref[...])


def _tc_mlp(embs, W1, b1, W2, b2):
    grid = (_B // _BB,)
    outp = pl.pallas_call(
        _mlp_body,
        grid=grid,
        in_specs=[pl.BlockSpec((n, _BB // 4, 128), lambda i: (0, i, 0))
                  for n in _CHUNKS] + [
            pl.BlockSpec((_H, _NF * _D), lambda i: (0, 0)),
            pl.BlockSpec((_H, 1), lambda i: (0, 0)),
            pl.BlockSpec((_O, _H), lambda i: (0, 0)),
            pl.BlockSpec((_O, 1), lambda i: (0, 0)),
        ],
        out_specs=pl.BlockSpec((1, _BB), lambda i: (0, i)),
        out_shape=jax.ShapeDtypeStruct((1, _B), jnp.float32),
    )(*embs, W1.T, b1.reshape(_H, 1), W2.T, b2.reshape(_O, 1))
    return outp.reshape(_B // _BB, 4, _BB // 4).transpose(0, 2, 1).reshape(_B, _O)


_TB = _V // 4


def _tr_body(t_ref, o_ref):
    s = t_ref[0]
    parts = [s[:, a * _TB:(a + 1) * _TB] for a in range(4)]
    stacked = jnp.concatenate(parts, axis=0)
    o_ref[...] = jnp.swapaxes(stacked, 0, 1)


def _tc_relayout(tabT_c, nf, lo=0):
    grid = (nf,)
    return pl.pallas_call(
        _tr_body,
        grid=grid,
        in_specs=[pl.BlockSpec((1, _D, _V), lambda f, lo=lo: (lo + f, 0, 0))],
        out_specs=pl.BlockSpec((_TB, 4 * _D), lambda f: (f, 0)),
        out_shape=jax.ShapeDtypeStruct((nf * _V // 4, 4 * _D), jnp.float32),
        compiler_params=pltpu.CompilerParams(vmem_limit_bytes=100 * 1024 * 1024),
    )(tabT_c)


def kernel(x, tables, W1, b1, W2, b2):
    f_local = np.concatenate([np.arange(n) for n in _CHUNKS])
    offs = jnp.asarray(f_local * _V, dtype=jnp.int32)[:, None]
    xi = x.astype(jnp.int32)
    perm = 4 * (xi % _TB) + xi // _TB
    idxT = perm.T + offs
    tabT = jnp.transpose(tables, (0, 2, 1))
    embs = []
    lo = 0
    for n, grp in zip(_CHUNKS, _GRPS):
        tf_c = _tc_relayout(tabT, n, lo)
        nrows = n * _B
        idx_c = idxT[lo:lo + n].reshape(_NW, nrows // (_NW * _CHUNK), _CHUNK)
        emb_c = _sc_gather(tf_c.reshape(n * _V, _D), idx_c, nrows, grp)
        embs.append(emb_c.reshape(n, _B * _D // 128, 128))
        lo += n
    return _tc_mlp(embs, W1, b1, W2, b2)


# speedup vs baseline: 1.8375x; 1.0004x over previous
import functools

import jax
import jax.numpy as jnp
import numpy as np
from jax import lax
from jax.experimental import pallas as pl
from jax.experimental.pallas import tpu as pltpu
from jax.experimental.pallas import tpu_sc as plsc

_B = 16384
_NF = 26
_V = 100000
_D = 32
_H = 256
_O = 1

_NC = 2
_NS = 16
_NW = _NC * _NS

_ROWS = _B * _NF
_ROWS_W = _ROWS // _NW
_CHUNK = 128
_NCHUNK = _ROWS_W // _CHUNK
_GROUP = 8
_GROUP_ROWS = _CHUNK * _GROUP
_NGROUP = _NCHUNK // _GROUP

_CHUNKS = (20, 6)
_GRPS = (16, 24)


def _make_gather_body(nchunk, ngroup, rows_w, group=_GROUP):
    def _gather_body(table_hbm, idx_hbm, out_hbm, idx_v, rows_v, sem):
        wid = lax.axis_index("s") * _NC + lax.axis_index("c")
        base = wid * rows_w
        pltpu.sync_copy(idx_hbm.at[wid], idx_v)

        grows = group * _CHUNK

        def body(g, carry):
            cbase = g * group
            copies = []
            for j in range(group):
                cp = pltpu.async_copy(
                    table_hbm.at[idx_v.at[cbase + j]],
                    rows_v.at[pl.ds(j * _CHUNK, _CHUNK)],
                    sem,
                )
                copies.append(cp)
            for cp in copies:
                cp.wait()
            pltpu.sync_copy(
                rows_v, out_hbm.at[pl.ds(base + g * grows, grows)]
            )
            return carry

        lax.fori_loop(0, ngroup, body, 0)

    return _gather_body


def _sc_gather(table_flat, idx3, nrows, group=_GROUP):
    rows_w = nrows // _NW
    nchunk = rows_w // _CHUNK
    ngroup = nchunk // group
    mesh = plsc.VectorSubcoreMesh(core_axis_name="c", subcore_axis_name="s")
    f = pl.kernel(
        _make_gather_body(nchunk, ngroup, rows_w, group),
        mesh=mesh,
        out_type=jax.ShapeDtypeStruct((nrows, _D), jnp.float32),
        scratch_types=[
            pltpu.VMEM((nchunk, _CHUNK), jnp.int32),
            pltpu.VMEM((group * _CHUNK, _D), jnp.float32),
            pltpu.SemaphoreType.DMA,
        ],
        compiler_params=pltpu.CompilerParams(use_tc_tiling_on_sc=False),
    )
    return f(table_flat, idx3)


_BB = 512


def _mlp_body(*refs):
    nch = len(_CHUNKS)
    e_refs = refs[:nch]
    w1t_ref, b1_ref, w2t_ref, b2_ref, out_ref = refs[nch:]
    pieces = []
    for c in range(nch):
        for j in range(_CHUNKS[c]):
            pjt = jnp.swapaxes(e_refs[c][j], 0, 1)
            pieces.append(jnp.concatenate(
                [pjt[32 * a:32 * (a + 1), :] for a in range(4)], axis=1))
    et = jnp.concatenate(pieces, axis=0).astype(jnp.bfloat16)
    h = jnp.dot(w1t_ref[...].astype(jnp.bfloat16), et,
                preferred_element_type=jnp.float32)
    h = jnp.maximum(h + b1_ref[...], 0.0)
    o = jnp.dot(w2t_ref[...], h, preferred_element_type=jnp.float32)
    out_ref[...] = jax.nn.sigmoid(o + b2_ref[...])


def _tc_mlp(embs, W1, b1, W2, b2):
    grid = (_B // _BB,)
    outp = pl.pallas_call(
        _mlp_body,
        grid=grid,
        in_specs=[pl.BlockSpec((n, _BB // 4, 128), lambda i: (0, i, 0))
                  for n in _CHUNKS] + [
            pl.BlockSpec((_H, _NF * _D), lambda i: (0, 0)),
            pl.BlockSpec((_H, 1), lambda i: (0, 0)),
            pl.BlockSpec((_O, _H), lambda i: (0, 0)),
            pl.BlockSpec((_O, 1), lambda i: (0, 0)),
        ],
        out_specs=pl.BlockSpec((1, _BB), lambda i: (0, i)),
        out_shape=jax.ShapeDtypeStruct((1, _B), jnp.float32),
    )(*embs, W1.T, b1.reshape(_H, 1), W2.T, b2.reshape(_O, 1))
    return outp.reshape(_B // _BB, 4, _BB // 4).transpose(0, 2, 1).reshape(_B, _O)


_TB = _V // 4


def _tr_body(t_ref, o_ref):
    s = t_ref[0]
    parts = [s[:, a * _TB:(a + 1) * _TB] for a in range(4)]
    stacked = jnp.concatenate(parts, axis=0)
    o_ref[...] = jnp.swapaxes(stacked, 0, 1)


def _tc_relayout(tabT_c, nf, lo=0):
    grid = (nf,)
    return pl.pallas_call(
        _tr_body,
        grid=grid,
        in_specs=[pl.BlockSpec((1, _D, _V), lambda f, lo=lo: (lo + f, 0, 0))],
        out_specs=pl.BlockSpec((_TB, 4 * _D), lambda f: (f, 0)),
        out_shape=jax.ShapeDtypeStruct((nf * _V // 4, 4 * _D), jnp.float32),
        compiler_params=pltpu.CompilerParams(vmem_limit_bytes=100 * 1024 * 1024),
    )(tabT_c)


def kernel(x, tables, W1, b1, W2, b2):
    f_local = np.concatenate([np.arange(n) for n in _CHUNKS])
    offs = jnp.asarray(f_local * _V, dtype=jnp.int32)[:, None]
    xi = x.astype(jnp.int32)
    perm = 4 * (xi % _TB) + xi // _TB
    idxT = perm.T + offs
    tabT = jnp.transpose(tables, (0, 2, 1))
    embs = []
    lo = 0
    for n, grp in zip(_CHUNKS, _GRPS):
        tf_c = _tc_relayout(tabT, n, lo)
        nrows = n * _B
        idx_c = idxT[lo:lo + n].reshape(_NW, nrows // (_NW * _CHUNK), _CHUNK)
        emb_c = _sc_gather(tf_c.reshape(n * _V, _D), idx_c, nrows, grp)
        embs.append(emb_c.reshape(n, _B * _D // 128, 128))
        lo += n
    return _tc_mlp(embs, W1, b1, W2, b2)


# speedup vs baseline: 1.8381x; 1.0003x over previous
import functools

import jax
import jax.numpy as jnp
import numpy as np
from jax import lax
from jax.experimental import pallas as pl
from jax.experimental.pallas import tpu as pltpu
from jax.experimental.pallas import tpu_sc as plsc

_B = 16384
_NF = 26
_V = 100000
_D = 32
_H = 256
_O = 1

_NC = 2
_NS = 16
_NW = _NC * _NS

_ROWS = _B * _NF
_ROWS_W = _ROWS // _NW
_CHUNK = 128
_NCHUNK = _ROWS_W // _CHUNK
_GROUP = 8
_GROUP_ROWS = _CHUNK * _GROUP
_NGROUP = _NCHUNK // _GROUP

_CHUNKS = (20, 6)
_GRPS = (16, 24)


def _make_gather_body(nchunk, ngroup, rows_w, group=_GROUP):
    def _gather_body(table_hbm, idx_hbm, out_hbm, idx_v, rows_v, sem):
        wid = lax.axis_index("s") * _NC + lax.axis_index("c")
        base = wid * rows_w
        pltpu.sync_copy(idx_hbm.at[wid], idx_v)

        grows = group * _CHUNK

        def body(g, carry):
            cbase = g * group
            copies = []
            for j in range(group):
                cp = pltpu.async_copy(
                    table_hbm.at[idx_v.at[cbase + j]],
                    rows_v.at[pl.ds(j * _CHUNK, _CHUNK)],
                    sem,
                )
                copies.append(cp)
            for cp in copies:
                cp.wait()
            pltpu.sync_copy(
                rows_v, out_hbm.at[pl.ds(base + g * grows, grows)]
            )
            return carry

        lax.fori_loop(0, ngroup, body, 0)

    return _gather_body


def _sc_gather(table_flat, idx3, nrows, group=_GROUP):
    rows_w = nrows // _NW
    nchunk = rows_w // _CHUNK
    ngroup = nchunk // group
    mesh = plsc.VectorSubcoreMesh(core_axis_name="c", subcore_axis_name="s")
    f = pl.kernel(
        _make_gather_body(nchunk, ngroup, rows_w, group),
        mesh=mesh,
        out_type=jax.ShapeDtypeStruct((nrows, _D), jnp.float32),
        scratch_types=[
            pltpu.VMEM((nchunk, _CHUNK), jnp.int32),
            pltpu.VMEM((group * _CHUNK, _D), jnp.float32),
            pltpu.SemaphoreType.DMA,
        ],
        compiler_params=pltpu.CompilerParams(use_tc_tiling_on_sc=False),
    )
    return f(table_flat, idx3)


_BB = 512


def _mlp_body(*refs):
    nch = len(_CHUNKS)
    e_refs = refs[:nch]
    w1t_ref, b1_ref, w2t_ref, b2_ref, out_ref = refs[nch:]
    pieces = []
    for c in range(nch):
        for j in range(_CHUNKS[c]):
            pjt = jnp.swapaxes(e_refs[c][j], 0, 1)
            pieces.append(jnp.concatenate(
                [pjt[32 * a:32 * (a + 1), :] for a in range(4)], axis=1))
    et = jnp.concatenate(pieces, axis=0)
    h = jnp.dot(w1t_ref[...], et, preferred_element_type=jnp.float32)
    h = jnp.maximum(h + b1_ref[...], 0.0)
    o = jnp.dot(w2t_ref[...], h, preferred_element_type=jnp.float32)
    out_ref[...] = jax.nn.sigmoid(o + b2_ref[...])


def _tc_mlp(embs, W1, b1, W2, b2):
    grid = (_B // _BB,)
    outp = pl.pallas_call(
        _mlp_body,
        grid=grid,
        in_specs=[pl.BlockSpec((n, _BB // 4, 128), lambda i: (0, i, 0))
                  for n in _CHUNKS] + [
            pl.BlockSpec((_H, _NF * _D), lambda i: (0, 0)),
            pl.BlockSpec((_H, 1), lambda i: (0, 0)),
            pl.BlockSpec((_O, _H), lambda i: (0, 0)),
            pl.BlockSpec((_O, 1), lambda i: (0, 0)),
        ],
        out_specs=pl.BlockSpec((1, _BB), lambda i: (0, i)),
        out_shape=jax.ShapeDtypeStruct((1, _B), jnp.float32),
    )(*embs, W1.T, b1.reshape(_H, 1), W2.T, b2.reshape(_O, 1))
    return outp.reshape(_B // _BB, 4, _BB // 4).transpose(0, 2, 1).reshape(_B, _O)


_TB = _V // 4


def _tr_body(t_ref, o_ref):
    s = t_ref[0]
    parts = [s[:, a * _TB:(a + 1) * _TB] for a in range(4)]
    stacked = jnp.concatenate(parts, axis=0)
    o_ref[...] = jnp.swapaxes(stacked, 0, 1)


def _tc_relayout(tabT_c, nf, lo=0):
    grid = (nf,)
    return pl.pallas_call(
        _tr_body,
        grid=grid,
        in_specs=[pl.BlockSpec((1, _D, _V), lambda f, lo=lo: (lo + f, 0, 0))],
        out_specs=pl.BlockSpec((_TB, 4 * _D), lambda f: (f, 0)),
        out_shape=jax.ShapeDtypeStruct((nf * _V // 4, 4 * _D), jnp.float32),
        compiler_params=pltpu.CompilerParams(vmem_limit_bytes=100 * 1024 * 1024),
    )(tabT_c)


def kernel(x, tables, W1, b1, W2, b2):
    f_local = np.concatenate([np.arange(n) for n in _CHUNKS])
    offs = jnp.asarray(f_local * _V, dtype=jnp.int32)[:, None]
    xi = x.astype(jnp.int32)
    perm = 4 * (xi % _TB) + xi // _TB
    idxT = perm.T + offs
    tabT = jnp.transpose(tables, (0, 2, 1))
    embs = []
    lo = 0
    for n, grp in zip(_CHUNKS, _GRPS):
        tf_c = _tc_relayout(tabT, n, lo)
        nrows = n * _B
        idx_c = idxT[lo:lo + n].reshape(_NW, nrows // (_NW * _CHUNK), _CHUNK)
        emb_c = _sc_gather(tf_c.reshape(n * _V, _D), idx_c, nrows, grp)
        embs.append(emb_c.reshape(n, _B * _D // 128, 128))
        lo += n
    return _tc_mlp(embs, W1, b1, W2, b2)
